# baseline (speedup 1.0000x reference)
import functools

import jax
import jax.numpy as jnp
from jax import lax
from jax.experimental import pallas as pl
from jax.experimental.pallas import tpu as pltpu
from jax.experimental.pallas import tpu_sc as plsc

_C = 19
_B, _H, _W = 4, 512, 512
_HB = 64
_NPIX = _B * _H * _W

_NW = 32
_CH = _NPIX // _NW
_LB = 16 * _C
_LBP = 512


def _sc_hist_kernel(t_hbm, out_hbm, tv, binsv):
    wid = lax.axis_index("s") * 2 + lax.axis_index("c")
    pltpu.sync_copy(t_hbm.at[pl.ds(pl.multiple_of(wid * _CH, _CH), _CH)], tv)

    lane = lax.iota(jnp.int32, 16)
    ones = jnp.ones((16,), jnp.int32)
    for j in range(_LBP // 16):
        binsv[pl.ds(j * 16, 16)] = jnp.zeros((16,), jnp.int32)

    def body(i, carry):
        v = tv[pl.ds(i * 16, 16)]
        plsc.addupdate_scatter(binsv, [v * 16 + lane], ones)
        return carry

    lax.fori_loop(0, _CH // 16, body, 0)
    pltpu.sync_copy(binsv, out_hbm.at[pl.ds(pl.multiple_of(wid * _LBP, _LBP),
                                            _LBP)])


_sc_hist = functools.partial(
    pl.kernel,
    out_type=jax.ShapeDtypeStruct((_NW * _LBP,), jnp.int32),
    mesh=plsc.VectorSubcoreMesh(core_axis_name="c", subcore_axis_name="s"),
    compiler_params=pltpu.CompilerParams(needs_layout_passes=False),
    scratch_types=[
        pltpu.VMEM((_CH,), jnp.int32),
        pltpu.VMEM((_LBP,), jnp.int32),
    ],
)(_sc_hist_kernel)


def _main_kernel(x_ref, t_ref, out_ref, acc_ref, acci_ref):
    step = pl.program_id(0)

    @pl.when(step == 0)
    def _init():
        acc_ref[...] = jnp.zeros_like(acc_ref)
        acci_ref[...] = jnp.zeros_like(acci_ref)

    t = t_ref[0]
    m = x_ref[0, 0]
    for c in range(1, _C):
        m = jnp.maximum(m, x_ref[0, c])
    first = jnp.full(t.shape, _C, jnp.int32)
    s = jnp.zeros_like(m)
    for c in range(_C - 1, -1, -1):
        v = x_ref[0, c]
        s = s + jnp.exp(v - m)
        first = jnp.where(v == m, c, first)
    lse = jnp.log(s) + m
    t16 = t.astype(jnp.int16)
    mism16 = (first != t).astype(jnp.int16)
    for c in range(_C):
        m16 = (t16 == c).astype(jnp.int16)
        acci_ref[c] += _rs16(m16 * mism16)
        acc_ref[c] += _rs(jnp.where(t == c, lse - x_ref[0, c], 0.0))

    @pl.when(step == pl.num_programs(0) - 1)
    def _fin():
        fn = jnp.sum(acci_ref[...].astype(jnp.float32), axis=(1, 2))
        ces = jnp.sum(acc_ref[...], axis=(1, 2))
        out_ref[...] = jnp.stack([fn, ces])


def _rs(a):
    return jnp.sum(a.reshape(_HB // 8, 8, _W), axis=0)


def _rs16(a):
    g = a.reshape(_HB // 8, 8, _W)
    r = g[0]
    for i in range(1, _HB // 8):
        r = r + g[i]
    return r


def _combine_kernel(gtl_ref, fnce_ref, out_ref):
    gt = jnp.sum(gtl_ref[...].astype(jnp.float32), axis=(0, 2))
    fn = fnce_ref[0]
    ces = fnce_ref[1]
    w = jnp.where(fn > 0, fn, 1.0) / jnp.where(gt > 0, gt, 1.0)
    out_ref[...] = jnp.reshape(jnp.sum(w * ces) / _NPIX, (1, 1))


def kernel(input, target):
    gt_lanes = _sc_hist(target.reshape(-1))
    nh = _H // _HB
    grid = (_B * nh,)
    fnce = pl.pallas_call(
        _main_kernel,
        grid=grid,
        in_specs=[
            pl.BlockSpec((1, _C, _HB, _W), lambda i: (i // nh, 0, i % nh, 0)),
            pl.BlockSpec((1, _HB, _W), lambda i: (i // nh, i % nh, 0)),
        ],
        out_specs=pl.BlockSpec((2, _C), lambda i: (0, 0)),
        out_shape=jax.ShapeDtypeStruct((2, _C), jnp.float32),
        scratch_shapes=[pltpu.VMEM((_C, 8, _W), jnp.float32),
                        pltpu.VMEM((_C, 8, _W), jnp.int16)],
        compiler_params=pltpu.CompilerParams(
            dimension_semantics=("arbitrary",),
        ),
    )(input, target)
    out = pl.pallas_call(
        _combine_kernel,
        out_shape=jax.ShapeDtypeStruct((1, 1), jnp.float32),
    )(gt_lanes.reshape(_NW, _LBP)[:, :_LB].reshape(_NW, _C, 16), fnce)
    return out[0, 0]


# speedup vs baseline: 1.0761x; 1.0761x over previous
import functools

import jax
import jax.numpy as jnp
from jax import lax
from jax.experimental import pallas as pl
from jax.experimental.pallas import tpu as pltpu
from jax.experimental.pallas import tpu_sc as plsc

_C = 19
_B, _H, _W = 4, 512, 512
_HB = 64
_NPIX = _B * _H * _W

_NW = 32
_CH = _NPIX // _NW
_LB = 16 * _C
_LBP = 512


_RW = 64


def _sc_hist_kernel(t_hbm, out_hbm, tv, binsv):
    wid = lax.axis_index("s") * 2 + lax.axis_index("c")
    b = wid // 8
    r0 = pl.multiple_of((wid % 8) * _RW, _RW)
    pltpu.sync_copy(t_hbm.at[b, pl.ds(r0, _RW)], tv)

    lane = lax.iota(jnp.int32, 16)
    ones = jnp.ones((16,), jnp.int32)
    for j in range(_LBP // 16):
        binsv[pl.ds(j * 16, 16)] = jnp.zeros((16,), jnp.int32)

    def body(r, carry):
        for j in range(_W // 16):
            v = tv[r, pl.ds(j * 16, 16)]
            plsc.addupdate_scatter(binsv, [v * 16 + lane], ones)
        return carry

    lax.fori_loop(0, _RW, body, 0)
    pltpu.sync_copy(binsv, out_hbm.at[wid])


_sc_hist = functools.partial(
    pl.kernel,
    out_type=jax.ShapeDtypeStruct((_NW, _LBP), jnp.int32),
    mesh=plsc.VectorSubcoreMesh(core_axis_name="c", subcore_axis_name="s"),
    compiler_params=pltpu.CompilerParams(needs_layout_passes=False),
    scratch_types=[
        pltpu.VMEM((_RW, _W), jnp.int32),
        pltpu.VMEM((_LBP,), jnp.int32),
    ],
)(_sc_hist_kernel)


def _main_kernel(x_ref, t_ref, out_ref, acc_ref, acci_ref):
    step = pl.program_id(0)

    @pl.when(step == 0)
    def _init():
        acc_ref[...] = jnp.zeros_like(acc_ref)
        acci_ref[...] = jnp.zeros_like(acci_ref)

    t = t_ref[0]
    m = x_ref[0, 0]
    for c in range(1, _C):
        m = jnp.maximum(m, x_ref[0, c])
    first = jnp.full(t.shape, _C, jnp.int32)
    s = jnp.zeros_like(m)
    for c in range(_C - 1, -1, -1):
        v = x_ref[0, c]
        s = s + jnp.exp(v - m)
        first = jnp.where(v == m, c, first)
    lse = jnp.log(s) + m
    t16 = t.astype(jnp.int16)
    mism16 = (first != t).astype(jnp.int16)
    for c in range(_C):
        m16 = (t16 == c).astype(jnp.int16)
        acci_ref[c] += _rs16(m16 * mism16)
        acc_ref[c] += _rs(jnp.where(t == c, lse - x_ref[0, c], 0.0))

    @pl.when(step == pl.num_programs(0) - 1)
    def _fin():
        fn = jnp.sum(acci_ref[...].astype(jnp.float32), axis=(1, 2))
        ces = jnp.sum(acc_ref[...], axis=(1, 2))
        out_ref[...] = jnp.stack([fn, ces])


def _rs(a):
    return jnp.sum(a.reshape(_HB // 8, 8, _W), axis=0)


def _rs16(a):
    g = a.reshape(_HB // 8, 8, _W)
    r = g[0]
    for i in range(1, _HB // 8):
        r = r + g[i]
    return r


def _combine_kernel(gtl_ref, fnce_ref, out_ref):
    acc = jnp.float32(0.0)
    for c in range(_C):
        gt_c = jnp.sum(gtl_ref[:, c * 16:(c + 1) * 16].astype(jnp.float32))
        fn_c = fnce_ref[0, c]
        ce_c = fnce_ref[1, c]
        w_c = jnp.where(fn_c > 0, fn_c, 1.0) / jnp.where(gt_c > 0, gt_c, 1.0)
        acc = acc + w_c * ce_c
    out_ref[...] = jnp.reshape(acc / _NPIX, (1, 1))


def kernel(input, target):
    gt_lanes = _sc_hist(target)
    nh = _H // _HB
    grid = (_B * nh,)
    fnce = pl.pallas_call(
        _main_kernel,
        grid=grid,
        in_specs=[
            pl.BlockSpec((1, _C, _HB, _W), lambda i: (i // nh, 0, i % nh, 0)),
            pl.BlockSpec((1, _HB, _W), lambda i: (i // nh, i % nh, 0)),
        ],
        out_specs=pl.BlockSpec((2, _C), lambda i: (0, 0)),
        out_shape=jax.ShapeDtypeStruct((2, _C), jnp.float32),
        scratch_shapes=[pltpu.VMEM((_C, 8, _W), jnp.float32),
                        pltpu.VMEM((_C, 8, _W), jnp.int16)],
        compiler_params=pltpu.CompilerParams(
            dimension_semantics=("arbitrary",),
        ),
    )(input, target)
    out = pl.pallas_call(
        _combine_kernel,
        out_shape=jax.ShapeDtypeStruct((1, 1), jnp.float32),
    )(gt_lanes, fnce)
    return out[0, 0]


# speedup vs baseline: 1.0774x; 1.0012x over previous
import functools

import jax
import jax.numpy as jnp
from jax import lax
from jax.experimental import pallas as pl
from jax.experimental.pallas import tpu as pltpu
from jax.experimental.pallas import tpu_sc as plsc

_C = 19
_B, _H, _W = 4, 512, 512
_HB = 64
_NPIX = _B * _H * _W

_NW = 32
_CH = _NPIX // _NW
_LB = 16 * _C
_LBP = 512


_RW = 64


def _sc_hist_kernel(t_hbm, out_hbm, tv, binsv):
    wid = lax.axis_index("s") * 2 + lax.axis_index("c")
    b = wid // 8
    r0 = pl.multiple_of((wid % 8) * _RW, _RW)
    pltpu.sync_copy(t_hbm.at[b, pl.ds(r0, _RW)], tv)

    lane = lax.iota(jnp.int32, 16)
    ones = jnp.ones((16,), jnp.int32)
    for j in range(_LBP // 16):
        binsv[pl.ds(j * 16, 16)] = jnp.zeros((16,), jnp.int32)

    def body(i, carry):
        r = i // 4
        j0 = (i % 4) * 8
        for j in range(8):
            v = tv[r, pl.ds((j0 + j) * 16, 16)]
            plsc.addupdate_scatter(binsv, [v * 16 + lane], ones)
        return carry

    lax.fori_loop(0, _RW * 4, body, 0)
    pltpu.sync_copy(binsv, out_hbm.at[wid])


_sc_hist = functools.partial(
    pl.kernel,
    out_type=jax.ShapeDtypeStruct((_NW, _LBP), jnp.int32),
    mesh=plsc.VectorSubcoreMesh(core_axis_name="c", subcore_axis_name="s"),
    compiler_params=pltpu.CompilerParams(needs_layout_passes=False),
    scratch_types=[
        pltpu.VMEM((_RW, _W), jnp.int32),
        pltpu.VMEM((_LBP,), jnp.int32),
    ],
)(_sc_hist_kernel)


def _main_kernel(x_ref, t_ref, out_ref, acc_ref, acci_ref):
    step = pl.program_id(0)

    @pl.when(step == 0)
    def _init():
        acc_ref[...] = jnp.zeros_like(acc_ref)
        acci_ref[...] = jnp.zeros_like(acci_ref)

    t = t_ref[0]
    m = x_ref[0, 0]
    for c in range(1, _C):
        m = jnp.maximum(m, x_ref[0, c])
    first = jnp.full(t.shape, _C, jnp.int32)
    s = jnp.zeros_like(m)
    for c in range(_C - 1, -1, -1):
        v = x_ref[0, c]
        s = s + jnp.exp(v - m)
        first = jnp.where(v == m, c, first)
    lse = jnp.log(s) + m
    t16 = t.astype(jnp.int16)
    mism16 = (first != t).astype(jnp.int16)
    for c in range(_C):
        m16 = (t16 == c).astype(jnp.int16)
        acci_ref[c] += _rs16(m16 * mism16)
        acc_ref[c] += _rs(jnp.where(t == c, lse - x_ref[0, c], 0.0))

    @pl.when(step == pl.num_programs(0) - 1)
    def _fin():
        fn = jnp.sum(acci_ref[...].astype(jnp.float32), axis=(1, 2))
        ces = jnp.sum(acc_ref[...], axis=(1, 2))
        out_ref[...] = jnp.stack([fn, ces])


def _rs(a):
    return jnp.sum(a.reshape(_HB // 8, 8, _W), axis=0)


def _rs16(a):
    g = a.reshape(_HB // 8, 8, _W)
    r = g[0]
    for i in range(1, _HB // 8):
        r = r + g[i]
    return r


def _combine_kernel(gtl_ref, fnce_ref, out_ref):
    acc = jnp.float32(0.0)
    for c in range(_C):
        gt_c = jnp.sum(gtl_ref[:, c * 16:(c + 1) * 16].astype(jnp.float32))
        fn_c = fnce_ref[0, c]
        ce_c = fnce_ref[1, c]
        w_c = jnp.where(fn_c > 0, fn_c, 1.0) / jnp.where(gt_c > 0, gt_c, 1.0)
        acc = acc + w_c * ce_c
    out_ref[...] = jnp.reshape(acc / _NPIX, (1, 1))


def kernel(input, target):
    gt_lanes = _sc_hist(target)
    nh = _H // _HB
    grid = (_B * nh,)
    fnce = pl.pallas_call(
        _main_kernel,
        grid=grid,
        in_specs=[
            pl.BlockSpec((1, _C, _HB, _W), lambda i: (i // nh, 0, i % nh, 0)),
            pl.BlockSpec((1, _HB, _W), lambda i: (i // nh, i % nh, 0)),
        ],
        out_specs=pl.BlockSpec((2, _C), lambda i: (0, 0)),
        out_shape=jax.ShapeDtypeStruct((2, _C), jnp.float32),
        scratch_shapes=[pltpu.VMEM((_C, 8, _W), jnp.float32),
                        pltpu.VMEM((_C, 8, _W), jnp.int16)],
        compiler_params=pltpu.CompilerParams(
            dimension_semantics=("arbitrary",),
        ),
    )(input, target)
    out = pl.pallas_call(
        _combine_kernel,
        out_shape=jax.ShapeDtypeStruct((1, 1), jnp.float32),
    )(gt_lanes, fnce)
    return out[0, 0]
